# Initial kernel scaffold; baseline (speedup 1.0000x reference)
import functools

import jax
import jax.numpy as jnp
from jax import lax
from jax.experimental import pallas as pl
from jax.experimental.pallas import tpu as pltpu
from jax.experimental.pallas import tpu_sc as plsc

EMBED = 32
NC, NS = 2, 16
NW = NC * NS


@functools.lru_cache(maxsize=None)
def _make_gather(n_rows: int, chunk: int):
    b_per_w = n_rows // NW
    steps = b_per_w // chunk
    assert steps * chunk == b_per_w and b_per_w * NW == n_rows

    mesh = plsc.VectorSubcoreMesh(
        core_axis_name="c", subcore_axis_name="s",
        num_cores=NC, num_subcores=NS)

    @functools.partial(
        pl.kernel,
        out_type=jax.ShapeDtypeStruct((n_rows, EMBED), jnp.float32),
        mesh=mesh,
        scratch_types=[
            pltpu.VMEM((chunk,), jnp.int32),
            pltpu.VMEM((chunk, EMBED), jnp.float32),
            pltpu.SemaphoreType.DMA,
        ],
    )
    def grab(idx_hbm, tab_hbm, out_hbm, idx_v, rows_v, sem):
        wid = lax.axis_index("s") * NC + lax.axis_index("c")
        base = wid * b_per_w

        def body(g, carry):
            off = base + g * chunk
            pltpu.sync_copy(idx_hbm.at[pl.ds(off, chunk)], idx_v)
            pltpu.async_copy(tab_hbm.at[idx_v], rows_v, sem).wait()
            pltpu.sync_copy(rows_v, out_hbm.at[pl.ds(off, chunk)])
            return carry

        lax.fori_loop(0, steps, body, 0)

    return grab


def kernel(inputs, embeddings):
    batch, hist = inputs.shape
    n_rows = batch * hist
    idx = inputs.reshape(n_rows).astype(jnp.int32)
    out = _make_gather(n_rows, 3200)(idx, embeddings)
    return out.reshape(batch, hist, EMBED)


# speedup vs baseline: 1.1112x; 1.1112x over previous
import functools

import jax
import jax.numpy as jnp
from jax import lax
from jax.experimental import pallas as pl
from jax.experimental.pallas import tpu as pltpu
from jax.experimental.pallas import tpu_sc as plsc

EMBED = 32
NC, NS = 2, 16
NW = NC * NS


@functools.lru_cache(maxsize=None)
def _make_gather(n_rows: int, chunk: int):
    b_per_w = n_rows // NW
    steps = b_per_w // chunk
    assert steps * chunk == b_per_w and b_per_w * NW == n_rows

    mesh = plsc.VectorSubcoreMesh(
        core_axis_name="c", subcore_axis_name="s",
        num_cores=NC, num_subcores=NS)

    @functools.partial(
        pl.kernel,
        out_type=jax.ShapeDtypeStruct((n_rows, EMBED), jnp.float32),
        mesh=mesh,
        scratch_types=[
            pltpu.VMEM((chunk,), jnp.int32),
            pltpu.VMEM((chunk, EMBED), jnp.float32),
            pltpu.SemaphoreType.DMA,
        ],
        compiler_params=pltpu.CompilerParams(use_tc_tiling_on_sc=False),
    )
    def grab(idx_hbm, tab_hbm, out_hbm, idx_v, rows_v, sem):
        wid = lax.axis_index("s") * NC + lax.axis_index("c")
        base = wid * b_per_w

        def body(g, carry):
            off = base + g * chunk
            pltpu.sync_copy(idx_hbm.at[pl.ds(off, chunk)], idx_v)
            pltpu.async_copy(tab_hbm.at[idx_v], rows_v, sem).wait()
            pltpu.sync_copy(rows_v, out_hbm.at[pl.ds(off, chunk)])
            return carry

        lax.fori_loop(0, steps, body, 0)

    return grab


def kernel(inputs, embeddings):
    batch, hist = inputs.shape
    n_rows = batch * hist
    idx = inputs.reshape(n_rows).astype(jnp.int32)
    out = _make_gather(n_rows, 3200)(idx, embeddings)
    return out.reshape(batch, hist, EMBED)


# speedup vs baseline: 1.1130x; 1.0016x over previous
import functools

import jax
import jax.numpy as jnp
from jax import lax
from jax.experimental import pallas as pl
from jax.experimental.pallas import tpu as pltpu
from jax.experimental.pallas import tpu_sc as plsc

EMBED = 32
NC, NS = 2, 16
NW = NC * NS


@functools.lru_cache(maxsize=None)
def _make_gather(n_rows: int, chunk: int, nsub: int):
    b_per_w = n_rows // NW
    steps = b_per_w // chunk
    sub = chunk // nsub
    assert steps * chunk == b_per_w and b_per_w * NW == n_rows
    assert sub * nsub == chunk and sub % 8 == 0

    mesh = plsc.VectorSubcoreMesh(
        core_axis_name="c", subcore_axis_name="s",
        num_cores=NC, num_subcores=NS)

    @functools.partial(
        pl.kernel,
        out_type=jax.ShapeDtypeStruct((n_rows, EMBED), jnp.float32),
        mesh=mesh,
        scratch_types=[
            pltpu.VMEM((2, chunk), jnp.int32),
            pltpu.VMEM((2, chunk, EMBED), jnp.float32),
            pltpu.SemaphoreType.DMA,
            pltpu.SemaphoreType.DMA,
            pltpu.SemaphoreType.DMA,
            pltpu.SemaphoreType.DMA,
        ],
        compiler_params=pltpu.CompilerParams(use_tc_tiling_on_sc=False),
    )
    def grab(idx_hbm, tab_hbm, out_hbm, idx_v, rows_v, g0, g1, o0, o1):
        wid = lax.axis_index("s") * NC + lax.axis_index("c")
        base = wid * b_per_w
        gsem = (g0, g1)
        osem = (o0, o1)

        def load_idx(g, b):
            pltpu.sync_copy(idx_hbm.at[pl.ds(base + g * chunk, chunk)],
                            idx_v.at[b])

        def fire_gather(b):
            return [
                pltpu.async_copy(
                    tab_hbm.at[idx_v.at[b, pl.ds(s * sub, sub)]],
                    rows_v.at[b, pl.ds(s * sub, sub)],
                    gsem[b])
                for s in range(nsub)
            ]

        def fire_out(g, b):
            return pltpu.async_copy(
                rows_v.at[b],
                out_hbm.at[pl.ds(base + g * chunk, chunk)],
                osem[b])

        load_idx(0, 0)
        gds = {0: fire_gather(0)}
        ods = {}
        for g in range(steps):
            b = g % 2
            nb = (g + 1) % 2
            if g + 1 < steps:
                load_idx(g + 1, nb)
                if g >= 1:
                    ods.pop(nb).wait()
                gds[nb] = fire_gather(nb)
            for d in gds.pop(b):
                d.wait()
            ods[b] = fire_out(g, b)
        for d in ods.values():
            d.wait()

    return grab


def kernel(inputs, embeddings):
    batch, hist = inputs.shape
    n_rows = batch * hist
    idx = inputs.reshape(n_rows).astype(jnp.int32)
    out = _make_gather(n_rows, 1600, 4)(idx, embeddings)
    return out.reshape(batch, hist, EMBED)


# speedup vs baseline: 1.8034x; 1.6204x over previous
import functools

import jax
import jax.numpy as jnp
from jax import lax
from jax.experimental import pallas as pl
from jax.experimental.pallas import tpu as pltpu
from jax.experimental.pallas import tpu_sc as plsc

EMBED = 32
NC, NS = 2, 16
NW = NC * NS


@functools.lru_cache(maxsize=None)
def _make_gather(batch: int, hist: int, g_rows: int, nsub: int):
    rows_per_w = batch // NW
    steps = rows_per_w // g_rows
    chunk = g_rows * hist
    sub = chunk // nsub
    assert steps * g_rows == rows_per_w and rows_per_w * NW == batch
    assert sub * nsub == chunk and sub % 8 == 0

    mesh = plsc.VectorSubcoreMesh(
        core_axis_name="c", subcore_axis_name="s",
        num_cores=NC, num_subcores=NS)

    @functools.partial(
        pl.kernel,
        out_type=jax.ShapeDtypeStruct((batch, hist, EMBED), jnp.float32),
        mesh=mesh,
        scratch_types=[
            pltpu.VMEM((2, chunk), jnp.int32),
            pltpu.VMEM((2, chunk, EMBED), jnp.float32),
            pltpu.SemaphoreType.DMA,
            pltpu.SemaphoreType.DMA,
            pltpu.SemaphoreType.DMA,
            pltpu.SemaphoreType.DMA,
        ],
        compiler_params=pltpu.CompilerParams(use_tc_tiling_on_sc=False),
    )
    def grab(idx_hbm, tab_hbm, out_hbm, idx_v, rows_v, g0, g1, o0, o1):
        wid = lax.axis_index("s") * NC + lax.axis_index("c")
        base = wid * rows_per_w
        gsem = (g0, g1)
        osem = (o0, o1)

        def load_idx(g, b):
            pltpu.sync_copy(
                idx_hbm.at[pl.ds((base + g * g_rows) * hist, chunk)],
                idx_v.at[b])

        def fire_gather(b):
            return [
                pltpu.async_copy(
                    tab_hbm.at[idx_v.at[b, pl.ds(s * sub, sub)]],
                    rows_v.at[b, pl.ds(s * sub, sub)],
                    gsem[b])
                for s in range(nsub)
            ]

        def fire_out(g, b):
            return [
                pltpu.async_copy(
                    rows_v.at[b, pl.ds(r * hist, hist)],
                    out_hbm.at[base + g * g_rows + r],
                    osem[b])
                for r in range(g_rows)
            ]

        load_idx(0, 0)
        gds = {0: fire_gather(0)}
        ods = {}
        for g in range(steps):
            b = g % 2
            nb = (g + 1) % 2
            if g + 1 < steps:
                load_idx(g + 1, nb)
                if g >= 1:
                    for d in ods.pop(nb):
                        d.wait()
                gds[nb] = fire_gather(nb)
            for d in gds.pop(b):
                d.wait()
            ods[b] = fire_out(g, b)
        for ds in ods.values():
            for d in ds:
                d.wait()

    return grab


def kernel(inputs, embeddings):
    batch, hist = inputs.shape
    idx = inputs.reshape(batch * hist).astype(jnp.int32)
    return _make_gather(batch, hist, 32, 2)(idx, embeddings)
